# Initial kernel scaffold; baseline (speedup 1.0000x reference)
import jax
import jax.numpy as jnp
from jax.experimental import pallas as pl


def kernel(x_t, dx_dt, batch, reaction_index, W1, b1, W2, b2, W3, b3):
    raise NotImplementedError("write your pallas kernel here")


# speedup vs baseline: 7.3934x; 7.3934x over previous
import jax
import jax.numpy as jnp
from jax.experimental import pallas as pl

_B = 128
_T = 32
_APC = 32
_G = _B * _T
_N = _G * _APC
_H = 64
_BETA = 20.0
_BLK = 8192
_RPB = _BLK // (_T * _APC)
_STEPS = _N // _BLK


def _loss_kernel(xT_ref, dxT_ref, w1_ref, b1_ref, w2_ref, b2_ref,
                 w3_ref, b3_ref, out_ref):
    i = pl.program_id(0)
    xb = xT_ref[...]
    h = jnp.tanh(jax.lax.dot_general(
        w1_ref[...], xb, (((1,), (0,)), ((), ())),
        preferred_element_type=jnp.float32) + b1_ref[...])
    h = jnp.tanh(jax.lax.dot_general(
        w2_ref[...], h, (((1,), (0,)), ((), ())),
        preferred_element_type=jnp.float32) + b2_ref[...])
    ae = jnp.sum(h * w3_ref[...], axis=0, keepdims=True) + b3_ref[0, 0]
    dxb = dxT_ref[...]
    dsq = jnp.sum(dxb * dxb, axis=0, keepdims=True)

    arow = ae.reshape(_RPB, _T * _APC)
    drow = dsq.reshape(_RPB, _T * _APC)
    lidx = jax.lax.broadcasted_iota(jnp.int32, (_T * _APC, _T), 0)
    cidx = jax.lax.broadcasted_iota(jnp.int32, (_T * _APC, _T), 1)
    seg = (lidx // _APC == cidx).astype(jnp.float32)
    energy = jax.lax.dot_general(
        arow, seg, (((1,), (0,)), ((), ())),
        preferred_element_type=jnp.float32)
    vsum = jax.lax.dot_general(
        drow, seg, (((1,), (0,)), ((), ())),
        preferred_element_type=jnp.float32)

    lse_args = _BETA * energy + 0.5 * jnp.log(vsum)
    m = jnp.max(lse_args, axis=1, keepdims=True)
    lse = m + jnp.log(jnp.sum(jnp.exp(lse_args - m), axis=1, keepdims=True))
    part = (jnp.sum(lse) / (_B * _BETA)).reshape(1, 1)

    @pl.when(i == 0)
    def _():
        out_ref[...] = part

    @pl.when(i > 0)
    def _():
        out_ref[...] += part


def kernel(x_t, dx_dt, batch, reaction_index, W1, b1, W2, b2, W3, b3):
    xT = x_t.T
    dxT = dx_dt.T
    out = pl.pallas_call(
        _loss_kernel,
        grid=(_STEPS,),
        in_specs=[
            pl.BlockSpec((3, _BLK), lambda i: (0, i)),
            pl.BlockSpec((3, _BLK), lambda i: (0, i)),
            pl.BlockSpec((_H, 3), lambda i: (0, 0)),
            pl.BlockSpec((_H, 1), lambda i: (0, 0)),
            pl.BlockSpec((_H, _H), lambda i: (0, 0)),
            pl.BlockSpec((_H, 1), lambda i: (0, 0)),
            pl.BlockSpec((_H, 1), lambda i: (0, 0)),
            pl.BlockSpec((1, 1), lambda i: (0, 0)),
        ],
        out_specs=pl.BlockSpec((1, 1), lambda i: (0, 0)),
        out_shape=jax.ShapeDtypeStruct((1, 1), jnp.float32),
    )(xT, dxT, W1.T, b1[:, None], W2.T, b2[:, None], W3, b3[:, None])
    return out[0, 0]


# speedup vs baseline: 9.2841x; 1.2557x over previous
import jax
import jax.numpy as jnp
from jax.experimental import pallas as pl

_B = 128
_T = 32
_APC = 32
_G = _B * _T
_N = _G * _APC
_H = 64
_BETA = 20.0
_BLK = 32768
_RPB = _BLK // (_T * _APC)
_STEPS = _N // _BLK
_RA = _T * _APC


def _loss_kernel(x4_ref, dxT_ref, w1_ref, w2_ref, b2_ref, w3_ref, b3_ref,
                 out_ref):
    i = pl.program_id(0)
    h = jnp.tanh(jax.lax.dot_general(
        w1_ref[...], x4_ref[...], (((1,), (0,)), ((), ())),
        preferred_element_type=jnp.float32))
    h = jnp.tanh(jax.lax.dot_general(
        w2_ref[...], h, (((1,), (0,)), ((), ())),
        preferred_element_type=jnp.float32) + b2_ref[...])
    ae = jax.lax.dot_general(
        w3_ref[...], h, (((0,), (0,)), ((), ())),
        preferred_element_type=jnp.float32) + b3_ref[0, 0]
    dxb = dxT_ref[...]
    dsq = jnp.sum(dxb * dxb, axis=0, keepdims=True)

    both = jnp.concatenate([ae, dsq], axis=0).reshape(2 * _RPB, _RA)
    lidx = jax.lax.broadcasted_iota(jnp.int32, (_RA, _T), 0)
    cidx = jax.lax.broadcasted_iota(jnp.int32, (_RA, _T), 1)
    seg = (lidx // _APC == cidx).astype(jnp.float32)
    ev = jax.lax.dot_general(
        both, seg, (((1,), (0,)), ((), ())),
        preferred_element_type=jnp.float32)
    energy = ev[:_RPB]
    vsum = ev[_RPB:]

    lse_args = _BETA * energy + 0.5 * jnp.log(vsum)
    m = jnp.max(lse_args, axis=1, keepdims=True)
    lse = m + jnp.log(jnp.sum(jnp.exp(lse_args - m), axis=1, keepdims=True))
    part = (jnp.sum(lse) / (_B * _BETA)).reshape(1, 1)

    @pl.when(i == 0)
    def _():
        out_ref[...] = part

    @pl.when(i > 0)
    def _():
        out_ref[...] += part


def kernel(x_t, dx_dt, batch, reaction_index, W1, b1, W2, b2, W3, b3):
    ones = jnp.ones((_N, 1), dtype=x_t.dtype)
    x4 = jnp.concatenate([x_t, ones], axis=1).T
    dxT = dx_dt.T
    w1 = jnp.concatenate([W1, b1[None, :]], axis=0).T
    out = pl.pallas_call(
        _loss_kernel,
        grid=(_STEPS,),
        in_specs=[
            pl.BlockSpec((4, _BLK), lambda i: (0, i)),
            pl.BlockSpec((3, _BLK), lambda i: (0, i)),
            pl.BlockSpec((_H, 4), lambda i: (0, 0)),
            pl.BlockSpec((_H, _H), lambda i: (0, 0)),
            pl.BlockSpec((_H, 1), lambda i: (0, 0)),
            pl.BlockSpec((_H, 1), lambda i: (0, 0)),
            pl.BlockSpec((1, 1), lambda i: (0, 0)),
        ],
        out_specs=pl.BlockSpec((1, 1), lambda i: (0, 0)),
        out_shape=jax.ShapeDtypeStruct((1, 1), jnp.float32),
    )(x4, dxT, w1, W2.T, b2[:, None], W3, b3[:, None])
    return out[0, 0]


# speedup vs baseline: 10.1628x; 1.0946x over previous
import jax
import jax.numpy as jnp
from jax import lax
from jax.experimental import pallas as pl

_B = 128
_T = 32
_APC = 32
_G = _B * _T
_N = _G * _APC
_H = 64
_BETA = 20.0
_BLK = 65536
_RA = _T * _APC
_RPB = _BLK // _RA
_STEPS = _N // _BLK


def _loss_kernel(x_ref, dx_ref, w1_ref, b1_ref, w2_ref, b2_ref, w3_ref,
                 b3_ref, out_ref):
    i = pl.program_id(0)
    h = jnp.tanh(lax.dot_general(
        w1_ref[...], x_ref[...], (((1,), (0,)), ((), ())),
        preferred_element_type=jnp.float32) + b1_ref[...])
    h = jnp.tanh(lax.dot_general(
        w2_ref[...], h, (((1,), (0,)), ((), ())),
        preferred_element_type=jnp.float32) + b2_ref[...])
    ae = jnp.sum(h * w3_ref[...], axis=0, keepdims=True) + b3_ref[0, 0]
    dxb = dx_ref[...]
    dsq = jnp.sum(dxb * dxb, axis=0, keepdims=True)

    arow = ae.reshape(_RPB, _RA)
    drow = dsq.reshape(_RPB, _RA)
    lidx = lax.broadcasted_iota(jnp.int32, (_RA, _T), 0)
    cidx = lax.broadcasted_iota(jnp.int32, (_RA, _T), 1)
    seg = (lidx // _APC == cidx).astype(jnp.float32)
    energy = lax.dot_general(arow, seg, (((1,), (0,)), ((), ())),
                             preferred_element_type=jnp.float32)
    vsum = lax.dot_general(drow, seg, (((1,), (0,)), ((), ())),
                           preferred_element_type=jnp.float32)

    lse_args = _BETA * energy + 0.5 * jnp.log(vsum)
    m = jnp.max(lse_args, axis=1, keepdims=True)
    lse = m + jnp.log(jnp.sum(jnp.exp(lse_args - m), axis=1, keepdims=True))
    part = (jnp.sum(lse) / (_B * _BETA)).reshape(1, 1)

    @pl.when(i == 0)
    def _():
        out_ref[...] = part

    @pl.when(i > 0)
    def _():
        out_ref[...] += part


def kernel(x_t, dx_dt, batch, reaction_index, W1, b1, W2, b2, W3, b3):
    out = pl.pallas_call(
        _loss_kernel,
        grid=(_STEPS,),
        in_specs=[
            pl.BlockSpec((3, _BLK), lambda i: (0, i)),
            pl.BlockSpec((3, _BLK), lambda i: (0, i)),
            pl.BlockSpec((_H, 3), lambda i: (0, 0)),
            pl.BlockSpec((_H, 1), lambda i: (0, 0)),
            pl.BlockSpec((_H, _H), lambda i: (0, 0)),
            pl.BlockSpec((_H, 1), lambda i: (0, 0)),
            pl.BlockSpec((_H, 1), lambda i: (0, 0)),
            pl.BlockSpec((1, 1), lambda i: (0, 0)),
        ],
        out_specs=pl.BlockSpec((1, 1), lambda i: (0, 0)),
        out_shape=jax.ShapeDtypeStruct((1, 1), jnp.float32),
    )(x_t.T, dx_dt.T, W1.T, b1[:, None], W2.T, b2[:, None], W3, b3[:, None])
    return out[0, 0]


# speedup vs baseline: 10.5197x; 1.0351x over previous
import jax
import jax.numpy as jnp
from jax import lax
from jax.experimental import pallas as pl

_B = 128
_T = 32
_APC = 32
_G = _B * _T
_N = _G * _APC
_H = 64
_BETA = 20.0
_BLK = 65536
_RA = _T * _APC
_RPB = _BLK // _RA
_STEPS = _N // _BLK


def _loss_kernel(x_ref, dx_ref, w1_ref, b1_ref, w2_ref, b2_ref, w3_ref,
                 b3_ref, out_ref):
    i = pl.program_id(0)
    h = jnp.tanh(lax.dot_general(
        w1_ref[...], x_ref[...], (((1,), (0,)), ((), ())),
        preferred_element_type=jnp.float32) + b1_ref[...])
    h = jnp.tanh(lax.dot_general(
        w2_ref[...], h, (((1,), (0,)), ((), ())),
        preferred_element_type=jnp.float32) + b2_ref[...])
    ae = lax.dot_general(w3_ref[...], h, (((0,), (0,)), ((), ())),
                         preferred_element_type=jnp.float32) + b3_ref[0, 0]
    dxb = dx_ref[...]
    dsq = jnp.sum(dxb * dxb, axis=0, keepdims=True)

    both = jnp.concatenate([ae, dsq], axis=0).reshape(2 * _RPB, _RA)
    lidx = lax.broadcasted_iota(jnp.int32, (_RA, _T), 0)
    cidx = lax.broadcasted_iota(jnp.int32, (_RA, _T), 1)
    seg = (lidx // _APC == cidx).astype(jnp.float32)
    ev = lax.dot_general(both, seg, (((1,), (0,)), ((), ())),
                         preferred_element_type=jnp.float32)
    energy = ev[:_RPB]
    vsum = ev[_RPB:]

    lse_args = _BETA * energy + 0.5 * jnp.log(vsum)
    m = jnp.max(lse_args, axis=1, keepdims=True)
    lse = m + jnp.log(jnp.sum(jnp.exp(lse_args - m), axis=1, keepdims=True))
    part = (jnp.sum(lse) / (_B * _BETA)).reshape(1, 1)

    @pl.when(i == 0)
    def _():
        out_ref[...] = part

    @pl.when(i > 0)
    def _():
        out_ref[...] += part


def kernel(x_t, dx_dt, batch, reaction_index, W1, b1, W2, b2, W3, b3):
    out = pl.pallas_call(
        _loss_kernel,
        grid=(_STEPS,),
        in_specs=[
            pl.BlockSpec((3, _BLK), lambda i: (0, i)),
            pl.BlockSpec((3, _BLK), lambda i: (0, i)),
            pl.BlockSpec((_H, 3), lambda i: (0, 0)),
            pl.BlockSpec((_H, 1), lambda i: (0, 0)),
            pl.BlockSpec((_H, _H), lambda i: (0, 0)),
            pl.BlockSpec((_H, 1), lambda i: (0, 0)),
            pl.BlockSpec((_H, 1), lambda i: (0, 0)),
            pl.BlockSpec((1, 1), lambda i: (0, 0)),
        ],
        out_specs=pl.BlockSpec((1, 1), lambda i: (0, 0)),
        out_shape=jax.ShapeDtypeStruct((1, 1), jnp.float32),
    )(x_t.T, dx_dt.T, W1.T, b1[:, None], W2.T, b2[:, None], W3, b3[:, None])
    return out[0, 0]
